# baseline (speedup 1.0000x reference)
import functools

import numpy as np
import jax
import jax.numpy as jnp
from jax import lax
from jax.experimental import pallas as pl
from jax.experimental.pallas import tpu as pltpu
from jax.experimental.pallas import tpu_sc as plsc

_B = 32
_N = 128
_C = 64
_P = _N * (_N + 1) // 2
_NC, _NS = 2, 16
_CHUNK = 128
_NFULL = _P // _CHUNK
_TAIL = _P - _NFULL * _CHUNK
_NCH = _NFULL + 1

_ti, _tj = np.tril_indices(_N)
_row = (_ti * _N + _tj).astype(np.int32)
_row = np.concatenate([_row, np.zeros(_NCH * _CHUNK - _P, np.int32)])
_GIDX = (_row[None, :] + (np.arange(_B, dtype=np.int32) * (_N * _N))[:, None]
         ).reshape(_B, _NCH, _CHUNK)


@functools.partial(
    pl.kernel,
    out_type=jax.ShapeDtypeStruct((_B * _P, _C), jnp.float32),
    mesh=plsc.VectorSubcoreMesh(core_axis_name="c", subcore_axis_name="s"),
    compiler_params=pltpu.CompilerParams(use_tc_tiling_on_sc=False),
    scratch_types=[
        pltpu.VMEM((_NCH, _CHUNK), jnp.int32),
        pltpu.VMEM((_CHUNK, _C), jnp.float32),
        pltpu.VMEM((_CHUNK, _C), jnp.float32),
        pltpu.SemaphoreType.DMA,
        pltpu.SemaphoreType.DMA,
        pltpu.SemaphoreType.DMA,
        pltpu.SemaphoreType.DMA,
    ],
)
def _tril_gather(rows_hbm, gidx_hbm, out_hbm,
                 idx_v, buf0, buf1, gs0, gs1, ws0, ws1):
    w = lax.axis_index("s") * _NC + lax.axis_index("c")
    pltpu.sync_copy(gidx_hbm.at[w], idx_v)
    base = w * _P
    bufs = (buf0, buf1)
    gsems = (gs0, gs1)
    wsems = (ws0, ws1)

    def gather(j, b):
        pltpu.async_copy(rows_hbm.at[idx_v.at[j]], bufs[b], gsems[b]).wait()

    def writeback(j, b):
        return pltpu.make_async_copy(
            bufs[b], out_hbm.at[pl.ds(base + j * _CHUNK, _CHUNK)], wsems[b])

    for b in range(2):
        gather(b, b)
        writeback(b, b).start()

    def paired(j2, _):
        for b in range(2):
            j = j2 * 2 + b
            writeback(j - 2, b).wait()
            gather(j, b)
            writeback(j, b).start()
        return ()

    lax.fori_loop(1, _NFULL // 2, paired, ())

    for b in range(2):
        writeback(_NFULL - 2 + b, b).wait()

    gather(_NFULL, 0)
    pltpu.sync_copy(buf0.at[pl.ds(0, _TAIL)],
                    out_hbm.at[pl.ds(base + _NFULL * _CHUNK, _TAIL)])


def kernel(x):
    rows = x.reshape(_B * _N * _N, _C)
    out = _tril_gather(rows, jnp.asarray(_GIDX))
    return out.reshape(_B, _P, _C)


# speedup vs baseline: 1.4214x; 1.4214x over previous
import functools

import numpy as np
import jax
import jax.numpy as jnp
from jax import lax
from jax.experimental import pallas as pl
from jax.experimental.pallas import tpu as pltpu
from jax.experimental.pallas import tpu_sc as plsc

_B = 32
_N = 128
_C = 64
_P = _N * (_N + 1) // 2
_NC, _NS = 2, 16
_CG = 4
_NCG = _C // _CG
_IQ = 32
_NQ = _N // _IQ

_ti, _tj = np.tril_indices(_N)
_TABLE = (((_ti % _IQ) << 7) | _tj).astype(np.int32)
_T32 = [0, 528, 2080, 4656, 8256]
_CHUNKS = [(_T32[q] // 16, _T32[q + 1] // 16) for q in range(_NQ)]


@functools.partial(
    pl.kernel,
    out_type=jax.ShapeDtypeStruct((_B * _C, _P), jnp.float32),
    mesh=plsc.VectorSubcoreMesh(core_axis_name="c", subcore_axis_name="s"),
    compiler_params=pltpu.CompilerParams(needs_layout_passes=False),
    scratch_types=[
        pltpu.VMEM((_P,), jnp.int32),
        pltpu.VMEM((_IQ, _CG, _N), jnp.float32),
        pltpu.VMEM((_IQ, _CG, _N), jnp.float32),
    ] + [pltpu.VMEM((_P,), jnp.float32) for _ in range(2 * _CG)] + [
        pltpu.SemaphoreType.DMA,
        pltpu.SemaphoreType.DMA,
        pltpu.SemaphoreType.DMA,
        pltpu.SemaphoreType.DMA,
    ],
)
def _tril_compact(xt_hbm, table_hbm, out_hbm, table_v, slab0, slab1, *rest):
    rows = (rest[:_CG], rest[_CG:2 * _CG])
    gsems = (rest[2 * _CG], rest[2 * _CG + 1])
    wsems = (rest[2 * _CG + 2], rest[2 * _CG + 3])
    slabs = (slab0, slab1)
    w = lax.axis_index("s") * _NC + lax.axis_index("c")
    pltpu.sync_copy(table_hbm, table_v)

    def stream(cg, q, sb):
        return pltpu.make_async_copy(
            xt_hbm.at[w, pl.ds(q * _IQ, _IQ), pl.ds(cg * _CG, _CG), :],
            slabs[sb], gsems[sb])

    def writeback(cg, par, cc):
        return pltpu.make_async_copy(
            rows[par][cc], out_hbm.at[w * _C + cg * _CG + cc], wsems[par])

    def compact(q, sb, par):
        k0, k1 = _CHUNKS[q]
        for cc in range(_CG):
            idx_c = jnp.full((16,), cc, jnp.int32)
            buf = rows[par][cc]

            def chunk(k, _, idx_c=idx_c, buf=buf, sb=sb):
                t = table_v[pl.ds(k * 16, 16)]
                vals = plsc.load_gather(
                    slabs[sb],
                    [lax.shift_right_logical(t, 7), idx_c,
                     lax.bitwise_and(t, 127)])
                buf[pl.ds(k * 16, 16)] = vals
                return ()

            lax.fori_loop(k0, k1, chunk, (), unroll=4)

    def group(cg, par):
        for q in range(_NQ):
            sb = q % 2
            stream(cg, q, sb).wait()
            nq, ncg = (q + 1, cg) if q + 1 < _NQ else (0, lax.rem(cg + 1, _NCG))
            stream(ncg, nq, 1 - sb).start()
            if q == 0:
                for cc in range(_CG):
                    writeback(cg, par, cc).wait()
            compact(q, sb, par)
        for cc in range(_CG):
            writeback(cg, par, cc).start()

    stream(0, 0, 0).start()
    for par in range(2):
        for cc in range(_CG):
            writeback(par, par, cc).start()

    def pair(g, _):
        group(2 * g, 0)
        group(2 * g + 1, 1)
        return ()

    lax.fori_loop(0, _NCG // 2, pair, ())

    stream(0, 0, 0).wait()
    for par in range(2):
        for cc in range(_CG):
            writeback(_NCG - 2 + par, par, cc).wait()


def kernel(x):
    xt = jnp.transpose(x, (0, 1, 3, 2))
    out2d = _tril_compact(xt, jnp.asarray(_TABLE))
    return jnp.transpose(out2d.reshape(_B, _C, _P), (0, 2, 1))
